# Initial kernel scaffold; baseline (speedup 1.0000x reference)
import jax
import jax.numpy as jnp
from jax.experimental import pallas as pl


def kernel(idx, targets, table):
    raise NotImplementedError("write your pallas kernel here")


# speedup vs baseline: 1.4745x; 1.4745x over previous
import functools

import jax
import jax.numpy as jnp
from jax import lax
from jax.experimental import pallas as pl
from jax.experimental.pallas import tpu as pltpu
from jax.experimental.pallas import tpu_sc as plsc

V = 1000
NTOK = 51200
NC, NS, L = 2, 16, 16
NW = NC * NS
PER_W = NTOK // NW
CH = 32
NCHUNK = PER_W // CH


def _logz_body(tab_ref, out_ref):
    x = tab_ref[...]
    m = jnp.max(x, axis=1, keepdims=True)
    e = jnp.exp(x - m)
    out_ref[...] = jnp.log(jnp.sum(e, axis=1, keepdims=True)) + m


_logz_call = pl.pallas_call(
    _logz_body,
    out_shape=jax.ShapeDtypeStruct((V, 1), jnp.float32),
)


def _loss_body(p_ref, out_ref):
    s = jnp.sum(p_ref[...]) * jnp.float32(1.0 / NTOK)
    out_ref[...] = jnp.full((1, 1), s, jnp.float32)


_loss_call = pl.pallas_call(
    _loss_body,
    out_shape=jax.ShapeDtypeStruct((1, 1), jnp.float32),
)


@functools.cache
def _make_sc_gather():
    mesh = plsc.VectorSubcoreMesh(core_axis_name="c", subcore_axis_name="s")
    return pl.kernel(
        _sc_gather_body,
        mesh=mesh,
        compiler_params=pltpu.CompilerParams(
            use_tc_tiling_on_sc=False, needs_layout_passes=False),
        out_type=[
            jax.ShapeDtypeStruct((NTOK, V), jnp.float32),
            jax.ShapeDtypeStruct((NW, L), jnp.float32),
        ],
        scratch_types=[
            pltpu.VMEM((PER_W,), jnp.int32),
            pltpu.VMEM((PER_W,), jnp.int32),
            pltpu.VMEM((V,), jnp.float32),
            pltpu.VMEM((L,), jnp.float32),
            pltpu.VMEM((CH, V), jnp.float32),
            pltpu.VMEM((CH, V), jnp.float32),
            pltpu.SemaphoreType.DMA,
            pltpu.SemaphoreType.DMA,
            pltpu.SemaphoreType.DMA,
            pltpu.SemaphoreType.DMA,
        ],
    )


def _sc_gather_body(table_hbm, idx_hbm, tgt_hbm, logz_hbm, out_hbm, part_hbm,
                    idx_v, tgt_v, logz_v, acc_v, buf0, buf1,
                    gs0, gs1, ss0, ss1):
    wid = lax.axis_index("s") * NC + lax.axis_index("c")
    base = wid * PER_W
    bufs = (buf0, buf1)
    gsems = (gs0, gs1)
    ssems = (ss0, ss1)

    pltpu.sync_copy(idx_hbm.at[pl.ds(base, PER_W)], idx_v)
    pltpu.sync_copy(tgt_hbm.at[pl.ds(base, PER_W)], tgt_v)
    pltpu.sync_copy(logz_hbm, logz_v)
    acc_v[...] = jnp.zeros((L,), jnp.float32)

    def _gather(c, b):
        pltpu.make_async_copy(
            table_hbm.at[idx_v.at[pl.ds(c * CH, CH)]], bufs[b], gsems[b]
        ).start()

    def _gather_wait(c, b):
        pltpu.make_async_copy(
            table_hbm.at[idx_v.at[pl.ds(c * CH, CH)]], bufs[b], gsems[b]
        ).wait()

    def _scatter(c, b):
        pltpu.make_async_copy(
            bufs[b], out_hbm.at[pl.ds(base + c * CH, CH)], ssems[b]
        ).start()

    def _scatter_wait(c, b):
        pltpu.make_async_copy(
            bufs[b], out_hbm.at[pl.ds(base + c * CH, CH)], ssems[b]
        ).wait()

    iota = lax.iota(jnp.int32, L)

    def _loss_update(c, b):
        for g in range(CH // L):
            off = c * CH + g * L
            rows = iota + g * L
            cols = tgt_v[pl.ds(off, L)]
            ii = idx_v[pl.ds(off, L)]
            vv = plsc.load_gather(bufs[b], [rows, cols])
            zz = plsc.load_gather(logz_v, [ii])
            acc_v[...] = acc_v[...] + (zz - vv)

    _gather(0, 0)

    def _pair(i, _):
        for b in range(2):
            c = i * 2 + b
            nb = 1 - b

            @pl.when(c + 1 < NCHUNK)
            def _():
                @pl.when(c >= 1)
                def _():
                    _scatter_wait(c - 1, nb)
                _gather(c + 1, nb)

            _gather_wait(c, b)
            _loss_update(c, b)
            _scatter(c, b)
        return ()

    lax.fori_loop(0, NCHUNK // 2, _pair, ())

    _scatter_wait(NCHUNK - 2, 0)
    _scatter_wait(NCHUNK - 1, 1)
    pltpu.sync_copy(acc_v, part_hbm.at[wid])


def kernel(idx, targets, table):
    idx_f = idx.reshape(-1).astype(jnp.int32)
    tgt_f = targets.reshape(-1).astype(jnp.int32)
    logz = _logz_call(table).reshape(-1)
    logits, parts = _make_sc_gather()(table, idx_f, tgt_f, logz)
    loss = _loss_call(parts)[0, 0]
    return logits, loss


# speedup vs baseline: 1.6119x; 1.0932x over previous
import functools

import jax
import jax.numpy as jnp
from jax import lax
from jax.experimental import pallas as pl
from jax.experimental.pallas import tpu as pltpu
from jax.experimental.pallas import tpu_sc as plsc

V = 1000
NTOK = 51200
NC, NS, L = 2, 16, 16
NW = NC * NS
PER_W = NTOK // NW
W = 16
NWIN = PER_W // W
TS = 24
C8 = V // 8
RB = NTOK // 128


def _logz_body(tab_ref, out_ref):
    x = tab_ref[...]
    m = jnp.max(x, axis=1, keepdims=True)
    e = jnp.exp(x - m)
    out_ref[...] = jnp.log(jnp.sum(e, axis=1, keepdims=True)) + m


_logz_call = pl.pallas_call(
    _logz_body,
    out_shape=jax.ShapeDtypeStruct((V, 1), jnp.float32),
)


def _loss_body(p_ref, out_ref):
    s = jnp.sum(p_ref[...]) * jnp.float32(1.0 / NTOK)
    out_ref[...] = jnp.full((1, 1), s, jnp.float32)


_loss_call = pl.pallas_call(
    _loss_body,
    out_shape=jax.ShapeDtypeStruct((1, 1), jnp.float32),
)


@functools.cache
def _make_sc_gather():
    mesh = plsc.VectorSubcoreMesh(core_axis_name="c", subcore_axis_name="s")
    return pl.kernel(
        _sc_gather_body,
        mesh=mesh,
        compiler_params=pltpu.CompilerParams(
            use_tc_tiling_on_sc=False, needs_layout_passes=False),
        out_type=[
            jax.ShapeDtypeStruct((C8, RB, 8, 128), jnp.float32),
            jax.ShapeDtypeStruct((NW, L), jnp.float32),
        ],
        scratch_types=[
            pltpu.VMEM((W,), jnp.int32),
            pltpu.VMEM((W,), jnp.int32),
            pltpu.VMEM((W,), jnp.int32),
            pltpu.VMEM((V,), jnp.float32),
            pltpu.VMEM((W, V), jnp.float32),
            pltpu.VMEM((W, V), jnp.float32),
            pltpu.VMEM((C8, 8, TS), jnp.float32),
            pltpu.VMEM((C8, 8, TS), jnp.float32),
            pltpu.VMEM((L,), jnp.float32),
            pltpu.SemaphoreType.DMA,
            pltpu.SemaphoreType.DMA,
            pltpu.SemaphoreType.DMA,
            pltpu.SemaphoreType.DMA,
            pltpu.SemaphoreType.DMA,
            pltpu.SemaphoreType.DMA,
            pltpu.SemaphoreType.DMA,
        ],
    )


def _sc_gather_body(table_hbm, idx_hbm, tgt_hbm, logz_hbm, out_hbm, part_hbm,
                    iw0, iw1, tw, logz_v, buf0, buf1, t0, t1, acc_v,
                    is0, is1, ts_sem, gs0, gs1, os0, os1):
    wid = lax.axis_index("s") * NC + lax.axis_index("c")
    base = wid * PER_W
    idxw = (iw0, iw1)
    bufs = (buf0, buf1)
    tbufs = (t0, t1)
    isems = (is0, is1)
    gsems = (gs0, gs1)
    osems = (os0, os1)

    iota = lax.iota(jnp.int32, L)
    civ = iota & 7
    step8 = iota >> 3

    pltpu.sync_copy(logz_hbm, logz_v)

    def _idx_copy(c, b):
        return pltpu.make_async_copy(
            idx_hbm.at[pl.ds(base + c * W, W)], idxw[b], isems[b])

    def _tgt_copy(c):
        return pltpu.make_async_copy(
            tgt_hbm.at[pl.ds(base + c * W, W)], tw, ts_sem)

    def _row_gather(b):
        return pltpu.make_async_copy(table_hbm.at[idxw[b]], bufs[b], gsems[b])

    def _out_dma(c, b):
        g0 = base + c * W
        rb = g0 // 128
        o = g0 % 128
        return pltpu.make_async_copy(
            tbufs[b].at[:, :, pl.ds(0, W)],
            out_hbm.at[:, rb, :, pl.ds(o, W)],
            osems[b],
        )

    def _transpose(b):
        buf = bufs[b]
        tb = tbufs[b]

        def _cstep(c, _):
            c8v = step8 + 2 * c
            for t in range(W):
                val = buf[t, pl.ds(c * 16, 16)]
                plsc.store_scatter(
                    tb, [c8v, civ, jnp.full((L,), t, jnp.int32)], val)
            return ()

        lax.fori_loop(0, 62, _cstep, ())
        c8v = step8 + 123
        for t in range(W):
            val = buf[t, pl.ds(984, 16)]
            plsc.store_scatter(
                tb, [c8v, civ, jnp.full((L,), t, jnp.int32)], val)

    def _loss_update(b, acc):
        rows = iota
        cols = tw[...]
        ii = idxw[b][...]
        vv = plsc.load_gather(bufs[b], [rows, cols])
        zz = plsc.load_gather(logz_v, [ii])
        return acc + (zz - vv)

    _idx_copy(0, 0).start()
    _tgt_copy(0).start()
    _idx_copy(1, 1).start()
    _idx_copy(0, 0).wait()
    _row_gather(0).start()

    def _pair(i, acc):
        for b in range(2):
            c = i * 2 + b
            nb = 1 - b

            _row_gather(b).wait()
            _tgt_copy(c).wait()
            acc = _loss_update(b, acc)

            @pl.when(c + 2 < NWIN)
            def _():
                _idx_copy(c + 2, b).start()

            @pl.when(c + 1 < NWIN)
            def _():
                _tgt_copy(c + 1).start()
                _idx_copy(c + 1, nb).wait()
                _row_gather(nb).start()

            @pl.when(c >= 2)
            def _():
                _out_dma(c - 2, b).wait()

            _transpose(b)
            _out_dma(c, b).start()
        return acc

    acc = lax.fori_loop(0, NWIN // 2, _pair,
                        jnp.zeros((L,), jnp.float32))

    _out_dma(NWIN - 2, 0).wait()
    _out_dma(NWIN - 1, 1).wait()
    acc_v[...] = acc
    pltpu.sync_copy(acc_v, part_hbm.at[wid])


def kernel(idx, targets, table):
    idx_f = idx.reshape(-1).astype(jnp.int32)
    tgt_f = targets.reshape(-1).astype(jnp.int32)
    logz = _logz_call(table).reshape(-1)
    logits_t, parts = _make_sc_gather()(table, idx_f, tgt_f, logz)
    logits = logits_t.transpose(1, 3, 0, 2).reshape(NTOK, V)
    loss = _loss_call(parts)[0, 0]
    return logits, loss
